# baseline (speedup 1.0000x reference)
import jax
import jax.numpy as jnp
from jax import lax
from jax.experimental import pallas as pl
from jax.experimental.pallas import tpu as pltpu

_R = 70
_H = 512
_S = 128
_STRIDE = 3
_CHUNK = 80
_CW = 256


def _propose_kernel(score_ref, fake_hbm, reala_hbm,
                    fbr_ref, rar_ref, fbf_ref, raf_ref, fabf_ref, rabr_ref,
                    buf_ref, sem):
    b = pl.program_id(0)
    s = score_ref[0, 0]
    ri = lax.broadcasted_iota(jnp.int32, (_S, _S), 0)
    ci = lax.broadcasted_iota(jnp.int32, (_S, _S), 1)
    flat = ri * _S + ci
    big = jnp.int32(1 << 30)
    vmax = jnp.max(s)
    vmin = jnp.min(s)
    imax = jnp.min(jnp.where(s == vmax, flat, big))
    imin = jnp.min(jnp.where(s == vmin, flat, big))
    rr = jnp.where(vmax > 0.0, imax // _S, 0) * _STRIDE + _R
    cr = jnp.where(vmax > 0.0, imax % _S, 0) * _STRIDE + _R
    rf = jnp.where(vmin < 1.0, imin // _S, 1) * _STRIDE + _R
    cf = jnp.where(vmin < 1.0, imin % _S, 1) * _STRIDE + _R

    def base(r0, c0):
        ra = jnp.minimum((r0 >> 3) << 3, _H - _CHUNK)
        ca = jnp.minimum((c0 >> 7) << 7, _H - _CW)
        return pl.multiple_of(ra, 8), pl.multiple_of(ca, 128)

    rar_b, car_b = base(rr, cr)
    raf_b, caf_b = base(rf, cf)

    plan = [(fake_hbm, rar_b, car_b), (reala_hbm, rar_b, car_b),
            (fake_hbm, raf_b, caf_b), (reala_hbm, raf_b, caf_b)]
    cps = []
    for i, (img, ra, ca) in enumerate(plan):
        cp = pltpu.make_async_copy(
            img.at[b, :, pl.ds(ra, _CHUNK), pl.ds(ca, _CW)],
            buf_ref.at[i], sem.at[i])
        cp.start()
        cps.append(cp)

    def selectors(r0, c0, ra, ca):
        jc = lax.broadcasted_iota(jnp.int32, (_CW, _R), 0)
        kc = lax.broadcasted_iota(jnp.int32, (_CW, _R), 1)
        csel = (ca + jc == c0 + kc).astype(jnp.float32)
        ir = lax.broadcasted_iota(jnp.int32, (_R, _CHUNK), 0)
        jr = lax.broadcasted_iota(jnp.int32, (_R, _CHUNK), 1)
        rsel = ((ra + jr == r0 + ir) & (r0 + ir <= _H - 1)).astype(jnp.float32)
        return csel, rsel

    csel_r, rsel_r = selectors(rr, cr, rar_b, car_b)
    csel_f, rsel_f = selectors(rf, cf, raf_b, caf_b)

    def crop(i, csel, rsel):
        cps[i].wait()
        outs = []
        for c in range(3):
            t = jnp.dot(buf_ref[i, c], csel, preferred_element_type=jnp.float32)
            outs.append(jnp.dot(rsel, t, preferred_element_type=jnp.float32))
        return outs

    fbr = crop(0, csel_r, rsel_r)
    rar = crop(1, csel_r, rsel_r)
    fbf = crop(2, csel_f, rsel_f)
    raf = crop(3, csel_f, rsel_f)
    for c in range(3):
        fbr_ref[0, c] = fbr[c]
        rar_ref[0, c] = rar[c]
        fbf_ref[0, c] = fbf[c]
        raf_ref[0, c] = raf[c]
        fabf_ref[0, c] = raf[c]
        fabf_ref[0, 3 + c] = fbf[c]
        rabr_ref[0, c] = rar[c]
        rabr_ref[0, 3 + c] = fbr[c]


def kernel(real_B, fake_B, real_A, score_map):
    del real_B
    B = score_map.shape[0]
    f32 = jnp.float32
    crop3 = jax.ShapeDtypeStruct((B, 3, _R, _R), f32)
    crop6 = jax.ShapeDtypeStruct((B, 6, _R, _R), f32)
    spec3 = pl.BlockSpec((1, 3, _R, _R), lambda b: (b, 0, 0, 0))
    spec6 = pl.BlockSpec((1, 6, _R, _R), lambda b: (b, 0, 0, 0))
    outs = pl.pallas_call(
        _propose_kernel,
        out_shape=(crop3, crop3, crop3, crop3, crop6, crop6),
        grid=(B,),
        in_specs=[
            pl.BlockSpec((1, 1, _S, _S), lambda b: (b, 0, 0, 0)),
            pl.BlockSpec(memory_space=pl.ANY),
            pl.BlockSpec(memory_space=pl.ANY),
        ],
        out_specs=(spec3, spec3, spec3, spec3, spec6, spec6),
        scratch_shapes=[
            pltpu.VMEM((4, 3, _CHUNK, _CW), f32),
            pltpu.SemaphoreType.DMA((4,)),
        ],
        compiler_params=pltpu.CompilerParams(
            dimension_semantics=("parallel",),
        ),
        name="drpan_proposal",
    )(score_map, fake_B, real_A)
    return tuple(outs)


# speedup vs baseline: 2.2327x; 2.2327x over previous
import jax
import jax.numpy as jnp
from jax import lax
from jax.experimental import pallas as pl
from jax.experimental.pallas import tpu as pltpu

_R = 70
_H = 512
_S = 128
_STRIDE = 3
_CHUNK = 80
_CW = 256
_GRID = 2


def _coord_kernel(score_ref, coord_ref):
    s = score_ref[:, 0]
    per = s.shape[0]
    ri = lax.broadcasted_iota(jnp.int32, (_S, _S), 0)
    ci = lax.broadcasted_iota(jnp.int32, (_S, _S), 1)
    flat = (ri * _S + ci)[None]
    big = jnp.int32(1 << 30)
    vmax = jnp.max(s, axis=(1, 2))
    vmin = jnp.min(s, axis=(1, 2))
    imax = jnp.min(jnp.where(s == vmax[:, None, None], flat, big), axis=(1, 2))
    imin = jnp.min(jnp.where(s == vmin[:, None, None], flat, big), axis=(1, 2))
    rr = jnp.where(vmax > 0.0, imax // _S, 0) * _STRIDE + _R
    cr = jnp.where(vmax > 0.0, imax % _S, 0) * _STRIDE + _R
    rf = jnp.where(vmin < 1.0, imin // _S, 1) * _STRIDE + _R
    cf = jnp.where(vmin < 1.0, imin % _S, 1) * _STRIDE + _R
    out = jnp.stack([rr, cr, rf, cf], axis=1)
    coord_ref[...] = jnp.concatenate(
        [out, jnp.zeros((per, 4), jnp.int32)], axis=1)


def _crops_kernel(coord_ref, fake_hbm, reala_hbm,
                  fbr_ref, rar_ref, fbf_ref, raf_ref, fabf_ref, rabr_ref,
                  buf_ref, sem):
    g = pl.program_id(0)
    per = fbr_ref.shape[0]

    def coords(j):
        bb = g * per + j
        return (coord_ref[bb, 0], coord_ref[bb, 1],
                coord_ref[bb, 2], coord_ref[bb, 3])

    def bases(r0, c0):
        ra = jnp.minimum((r0 >> 3) << 3, _H - _CHUNK)
        ca = jnp.minimum((c0 >> 7) << 7, _H - _CW)
        return pl.multiple_of(ra, 8), pl.multiple_of(ca, 128)

    def issue(j, slot):
        bb = g * per + j
        rr, cr, rf, cf = coords(j)
        cps = []
        for cs, (r0, c0) in enumerate(((rr, cr), (rf, cf))):
            ra, ca = bases(r0, c0)
            for im, img in enumerate((fake_hbm, reala_hbm)):
                for c in range(3):
                    cp = pltpu.make_async_copy(
                        img.at[bb, c, pl.ds(ra, _CHUNK), pl.ds(ca, _CW)],
                        buf_ref.at[slot, cs,
                                   pl.ds(im * 3 * _CHUNK + c * _CHUNK, _CHUNK)],
                        sem.at[slot, cs, im])
                    cp.start()
                    cps.append(cp)
        return cps

    pending = [None] * per
    pending[0] = issue(0, 0)
    for j in range(per):
        slot = j % 2
        if j + 1 < per:
            pending[j + 1] = issue(j + 1, (j + 1) % 2)
        for cp in pending[j]:
            cp.wait()
        rr, cr, rf, cf = coords(j)
        crops = []
        for cs, (r0, c0) in enumerate(((rr, cr), (rf, cf))):
            ra, ca = bases(r0, c0)
            jc = lax.broadcasted_iota(jnp.int32, (_CW, _R), 0)
            kc = lax.broadcasted_iota(jnp.int32, (_CW, _R), 1)
            csel = (ca + jc == c0 + kc).astype(jnp.float32)
            ir = lax.broadcasted_iota(jnp.int32, (_R, _CHUNK), 0)
            jr = lax.broadcasted_iota(jnp.int32, (_R, _CHUNK), 1)
            rsel = ((ra + jr == r0 + ir)
                    & (r0 + ir <= _H - 1)).astype(jnp.float32)
            t = jnp.dot(buf_ref[slot, cs], csel,
                        preferred_element_type=jnp.float32)
            crops.append([
                jnp.dot(rsel, t[k * _CHUNK:(k + 1) * _CHUNK],
                        preferred_element_type=jnp.float32)
                for k in range(6)])
        fbr, rar = crops[0][:3], crops[0][3:]
        fbf, raf = crops[1][:3], crops[1][3:]
        for c in range(3):
            fbr_ref[j, c] = fbr[c]
            rar_ref[j, c] = rar[c]
            fbf_ref[j, c] = fbf[c]
            raf_ref[j, c] = raf[c]
            fabf_ref[j, c] = raf[c]
            fabf_ref[j, 3 + c] = fbf[c]
            rabr_ref[j, c] = rar[c]
            rabr_ref[j, 3 + c] = fbr[c]


def kernel(real_B, fake_B, real_A, score_map):
    del real_B
    B = score_map.shape[0]
    per = B // _GRID
    i32 = jnp.int32
    f32 = jnp.float32
    coords = pl.pallas_call(
        _coord_kernel,
        out_shape=jax.ShapeDtypeStruct((B, 8), i32),
        grid=(_GRID,),
        in_specs=[pl.BlockSpec((per, 1, _S, _S), lambda g: (g, 0, 0, 0))],
        out_specs=pl.BlockSpec((per, 8), lambda g: (g, 0)),
        compiler_params=pltpu.CompilerParams(
            dimension_semantics=("parallel",),
        ),
        name="drpan_coords",
    )(score_map)

    crop3 = jax.ShapeDtypeStruct((B, 3, _R, _R), f32)
    crop6 = jax.ShapeDtypeStruct((B, 6, _R, _R), f32)
    spec3 = pl.BlockSpec((per, 3, _R, _R), lambda g, coord: (g, 0, 0, 0))
    spec6 = pl.BlockSpec((per, 6, _R, _R), lambda g, coord: (g, 0, 0, 0))
    outs = pl.pallas_call(
        _crops_kernel,
        out_shape=(crop3, crop3, crop3, crop3, crop6, crop6),
        grid_spec=pltpu.PrefetchScalarGridSpec(
            num_scalar_prefetch=1,
            grid=(_GRID,),
            in_specs=[
                pl.BlockSpec(memory_space=pl.ANY),
                pl.BlockSpec(memory_space=pl.ANY),
            ],
            out_specs=(spec3, spec3, spec3, spec3, spec6, spec6),
            scratch_shapes=[
                pltpu.VMEM((2, 2, 2 * 3 * _CHUNK, _CW), f32),
                pltpu.SemaphoreType.DMA((2, 2, 2)),
            ],
        ),
        compiler_params=pltpu.CompilerParams(
            dimension_semantics=("parallel",),
        ),
        name="drpan_crops",
    )(coords, fake_B, real_A)
    return tuple(outs)


# speedup vs baseline: 2.5612x; 1.1471x over previous
import jax
import jax.numpy as jnp
from jax import lax
from jax.experimental import pallas as pl
from jax.experimental.pallas import tpu as pltpu

_R = 70
_H = 512
_S = 128
_STRIDE = 3
_CHUNK = 80
_CW = 256
_GRID = 2
_SLOTS = 4
_DEPTH = 3


def _propose_kernel(score_ref, fake_hbm, reala_hbm,
                    fbr_ref, rar_ref, fbf_ref, raf_ref, fabf_ref, rabr_ref,
                    buf_ref, sem):
    g = pl.program_id(0)
    per = fbr_ref.shape[0]

    s = score_ref[:, 0]
    ri = lax.broadcasted_iota(jnp.int32, (_S, _S), 0)
    ci = lax.broadcasted_iota(jnp.int32, (_S, _S), 1)
    flat = (ri * _S + ci)[None]
    big = jnp.int32(1 << 30)
    vmax = jnp.max(s, axis=(1, 2))
    vmin = jnp.min(s, axis=(1, 2))
    imax = jnp.min(jnp.where(s == vmax[:, None, None], flat, big), axis=(1, 2))
    imin = jnp.min(jnp.where(s == vmin[:, None, None], flat, big), axis=(1, 2))
    rr = jnp.where(vmax > 0.0, imax // _S, 0) * _STRIDE + _R
    cr = jnp.where(vmax > 0.0, imax % _S, 0) * _STRIDE + _R
    rf = jnp.where(vmin < 1.0, imin // _S, 1) * _STRIDE + _R
    cf = jnp.where(vmin < 1.0, imin % _S, 1) * _STRIDE + _R
    cm = jnp.stack([rr, cr, rf, cf], axis=0)
    coord = [[cm[q, j] for q in range(4)] for j in range(per)]

    def bases(r0, c0):
        ra = jnp.minimum((r0 >> 3) << 3, _H - _CHUNK)
        ca = jnp.minimum((c0 >> 7) << 7, _H - _CW)
        return pl.multiple_of(ra, 8), pl.multiple_of(ca, 128)

    def issue(j, slot):
        bb = g * per + j
        rr_, cr_, rf_, cf_ = coord[j]
        cps = []
        for cs, (r0, c0) in enumerate(((rr_, cr_), (rf_, cf_))):
            ra, ca = bases(r0, c0)
            for im, img in enumerate((fake_hbm, reala_hbm)):
                for c in range(3):
                    cp = pltpu.make_async_copy(
                        img.at[bb, c, pl.ds(ra, _CHUNK), pl.ds(ca, _CW)],
                        buf_ref.at[slot, cs,
                                   pl.ds(im * 3 * _CHUNK + c * _CHUNK, _CHUNK)],
                        sem.at[slot, cs, im])
                    cp.start()
                    cps.append(cp)
        return cps

    pending = [None] * per
    for k in range(min(_DEPTH, per)):
        pending[k] = issue(k, k % _SLOTS)
    for j in range(per):
        slot = j % _SLOTS
        if j + _DEPTH < per:
            pending[j + _DEPTH] = issue(j + _DEPTH, (j + _DEPTH) % _SLOTS)
        for cp in pending[j]:
            cp.wait()
        rr_, cr_, rf_, cf_ = coord[j]
        crops = []
        for cs, (r0, c0) in enumerate(((rr_, cr_), (rf_, cf_))):
            ra, ca = bases(r0, c0)
            jc = lax.broadcasted_iota(jnp.int32, (_CW, _R), 0)
            kc = lax.broadcasted_iota(jnp.int32, (_CW, _R), 1)
            csel = (ca + jc == c0 + kc).astype(jnp.float32)
            ir = lax.broadcasted_iota(jnp.int32, (_R, _CHUNK), 0)
            jr = lax.broadcasted_iota(jnp.int32, (_R, _CHUNK), 1)
            rsel = ((ra + jr == r0 + ir)
                    & (r0 + ir <= _H - 1)).astype(jnp.float32)
            t = jnp.dot(buf_ref[slot, cs], csel,
                        preferred_element_type=jnp.float32)
            crops.append([
                jnp.dot(rsel, t[k * _CHUNK:(k + 1) * _CHUNK],
                        preferred_element_type=jnp.float32)
                for k in range(6)])
        fbr, rar = crops[0][:3], crops[0][3:]
        fbf, raf = crops[1][:3], crops[1][3:]
        for c in range(3):
            fbr_ref[j, c] = fbr[c]
            rar_ref[j, c] = rar[c]
            fbf_ref[j, c] = fbf[c]
            raf_ref[j, c] = raf[c]
            fabf_ref[j, c] = raf[c]
            fabf_ref[j, 3 + c] = fbf[c]
            rabr_ref[j, c] = rar[c]
            rabr_ref[j, 3 + c] = fbr[c]


def kernel(real_B, fake_B, real_A, score_map):
    del real_B
    B = score_map.shape[0]
    per = B // _GRID
    f32 = jnp.float32
    crop3 = jax.ShapeDtypeStruct((B, 3, _R, _R), f32)
    crop6 = jax.ShapeDtypeStruct((B, 6, _R, _R), f32)
    spec3 = pl.BlockSpec((per, 3, _R, _R), lambda g: (g, 0, 0, 0))
    spec6 = pl.BlockSpec((per, 6, _R, _R), lambda g: (g, 0, 0, 0))
    outs = pl.pallas_call(
        _propose_kernel,
        out_shape=(crop3, crop3, crop3, crop3, crop6, crop6),
        grid=(_GRID,),
        in_specs=[
            pl.BlockSpec((per, 1, _S, _S), lambda g: (g, 0, 0, 0)),
            pl.BlockSpec(memory_space=pl.ANY),
            pl.BlockSpec(memory_space=pl.ANY),
        ],
        out_specs=(spec3, spec3, spec3, spec3, spec6, spec6),
        scratch_shapes=[
            pltpu.VMEM((_SLOTS, 2, 2 * 3 * _CHUNK, _CW), f32),
            pltpu.SemaphoreType.DMA((_SLOTS, 2, 2)),
        ],
        compiler_params=pltpu.CompilerParams(
            dimension_semantics=("parallel",),
        ),
        name="drpan_proposal",
    )(score_map, fake_B, real_A)
    return tuple(outs)


# speedup vs baseline: 2.5741x; 1.0050x over previous
import jax
import jax.numpy as jnp
from jax import lax
from jax.experimental import pallas as pl
from jax.experimental.pallas import tpu as pltpu

_R = 70
_H = 512
_S = 128
_STRIDE = 3
_CHUNK = 80
_CW = 256
_GRID = 2
_SLOTS = 6
_DEPTH = 5


def _propose_kernel(score_ref, fake_hbm, reala_hbm,
                    fbr_ref, rar_ref, fbf_ref, raf_ref, fabf_ref, rabr_ref,
                    buf_ref, sem):
    g = pl.program_id(0)
    per = fbr_ref.shape[0]

    s = score_ref[:, 0]
    ri = lax.broadcasted_iota(jnp.int32, (_S, _S), 0)
    ci = lax.broadcasted_iota(jnp.int32, (_S, _S), 1)
    flat = (ri * _S + ci)[None]
    big = jnp.int32(1 << 30)
    vmax = jnp.max(s, axis=(1, 2))
    vmin = jnp.min(s, axis=(1, 2))
    imax = jnp.min(jnp.where(s == vmax[:, None, None], flat, big), axis=(1, 2))
    imin = jnp.min(jnp.where(s == vmin[:, None, None], flat, big), axis=(1, 2))
    rr = jnp.where(vmax > 0.0, imax // _S, 0) * _STRIDE + _R
    cr = jnp.where(vmax > 0.0, imax % _S, 0) * _STRIDE + _R
    rf = jnp.where(vmin < 1.0, imin // _S, 1) * _STRIDE + _R
    cf = jnp.where(vmin < 1.0, imin % _S, 1) * _STRIDE + _R
    cm = jnp.stack([rr, cr, rf, cf], axis=0)
    coord = [[cm[q, j] for q in range(4)] for j in range(per)]

    def bases(r0, c0):
        ra = jnp.minimum((r0 >> 3) << 3, _H - _CHUNK)
        ca = jnp.minimum((c0 >> 7) << 7, _H - _CW)
        return pl.multiple_of(ra, 8), pl.multiple_of(ca, 128)

    def issue(j, slot):
        bb = g * per + j
        rr_, cr_, rf_, cf_ = coord[j]
        cps = []
        for cs, (r0, c0) in enumerate(((rr_, cr_), (rf_, cf_))):
            ra, ca = bases(r0, c0)
            for im, img in enumerate((fake_hbm, reala_hbm)):
                for c in range(3):
                    cp = pltpu.make_async_copy(
                        img.at[bb, c, pl.ds(ra, _CHUNK), pl.ds(ca, _CW)],
                        buf_ref.at[slot, cs,
                                   pl.ds(im * 3 * _CHUNK + c * _CHUNK, _CHUNK)],
                        sem.at[slot, cs, im])
                    cp.start()
                    cps.append(cp)
        return cps

    pending = [None] * per
    for k in range(min(_DEPTH, per)):
        pending[k] = issue(k, k % _SLOTS)
    for j in range(per):
        slot = j % _SLOTS
        if j + _DEPTH < per:
            pending[j + _DEPTH] = issue(j + _DEPTH, (j + _DEPTH) % _SLOTS)
        for cp in pending[j]:
            cp.wait()
        rr_, cr_, rf_, cf_ = coord[j]
        crops = []
        for cs, (r0, c0) in enumerate(((rr_, cr_), (rf_, cf_))):
            ra, ca = bases(r0, c0)
            jc = lax.broadcasted_iota(jnp.int32, (_CW, _R), 0)
            kc = lax.broadcasted_iota(jnp.int32, (_CW, _R), 1)
            csel = (ca + jc == c0 + kc).astype(jnp.float32)
            ir = lax.broadcasted_iota(jnp.int32, (_R, _CHUNK), 0)
            jr = lax.broadcasted_iota(jnp.int32, (_R, _CHUNK), 1)
            rsel = ((ra + jr == r0 + ir)
                    & (r0 + ir <= _H - 1)).astype(jnp.float32)
            t = jnp.dot(buf_ref[slot, cs], csel,
                        preferred_element_type=jnp.float32)
            crops.append([
                jnp.dot(rsel, t[k * _CHUNK:(k + 1) * _CHUNK],
                        preferred_element_type=jnp.float32)
                for k in range(6)])
        fbr, rar = crops[0][:3], crops[0][3:]
        fbf, raf = crops[1][:3], crops[1][3:]
        for c in range(3):
            fbr_ref[j, c] = fbr[c]
            rar_ref[j, c] = rar[c]
            fbf_ref[j, c] = fbf[c]
            raf_ref[j, c] = raf[c]
            fabf_ref[j, c] = raf[c]
            fabf_ref[j, 3 + c] = fbf[c]
            rabr_ref[j, c] = rar[c]
            rabr_ref[j, 3 + c] = fbr[c]


def kernel(real_B, fake_B, real_A, score_map):
    del real_B
    B = score_map.shape[0]
    per = B // _GRID
    f32 = jnp.float32
    crop3 = jax.ShapeDtypeStruct((B, 3, _R, _R), f32)
    crop6 = jax.ShapeDtypeStruct((B, 6, _R, _R), f32)
    spec3 = pl.BlockSpec((per, 3, _R, _R), lambda g: (g, 0, 0, 0))
    spec6 = pl.BlockSpec((per, 6, _R, _R), lambda g: (g, 0, 0, 0))
    outs = pl.pallas_call(
        _propose_kernel,
        out_shape=(crop3, crop3, crop3, crop3, crop6, crop6),
        grid=(_GRID,),
        in_specs=[
            pl.BlockSpec((per, 1, _S, _S), lambda g: (g, 0, 0, 0)),
            pl.BlockSpec(memory_space=pl.ANY),
            pl.BlockSpec(memory_space=pl.ANY),
        ],
        out_specs=(spec3, spec3, spec3, spec3, spec6, spec6),
        scratch_shapes=[
            pltpu.VMEM((_SLOTS, 2, 2 * 3 * _CHUNK, _CW), f32),
            pltpu.SemaphoreType.DMA((_SLOTS, 2, 2)),
        ],
        compiler_params=pltpu.CompilerParams(
            dimension_semantics=("parallel",),
        ),
        name="drpan_proposal",
    )(score_map, fake_B, real_A)
    return tuple(outs)


# speedup vs baseline: 2.5992x; 1.0098x over previous
import jax
import jax.numpy as jnp
from jax import lax
from jax.experimental import pallas as pl
from jax.experimental.pallas import tpu as pltpu

_R = 70
_H = 512
_S = 128
_STRIDE = 3
_CHUNK = 80
_CW = 256
_GRID = 2
_SLOTS = 6
_DEPTH = 5


def _propose_kernel(score_ref, fake_hbm, reala_hbm,
                    fbr_ref, rar_ref, fbf_ref, raf_ref, fabf_ref, rabr_ref,
                    buf_ref, sem):
    g = pl.program_id(0)
    per = fbr_ref.shape[0]

    s = score_ref[:, 0]
    ri = lax.broadcasted_iota(jnp.int32, (_S, _S), 0)
    ci = lax.broadcasted_iota(jnp.int32, (_S, _S), 1)
    flat = (ri * _S + ci)[None]
    big = jnp.int32(1 << 30)
    vmax = jnp.max(s, axis=(1, 2))
    vmin = jnp.min(s, axis=(1, 2))
    imax = jnp.min(jnp.where(s == vmax[:, None, None], flat, big), axis=(1, 2))
    imin = jnp.min(jnp.where(s == vmin[:, None, None], flat, big), axis=(1, 2))
    rr = jnp.where(vmax > 0.0, imax // _S, 0) * _STRIDE + _R
    cr = jnp.where(vmax > 0.0, imax % _S, 0) * _STRIDE + _R
    rf = jnp.where(vmin < 1.0, imin // _S, 1) * _STRIDE + _R
    cf = jnp.where(vmin < 1.0, imin % _S, 1) * _STRIDE + _R
    cm = jnp.stack([rr, cr, rf, cf], axis=0)
    coord = [[cm[q, j] for q in range(4)] for j in range(per)]

    def bases(r0, c0):
        ra = jnp.minimum((r0 >> 3) << 3, _H - _CHUNK)
        ca = jnp.minimum((c0 >> 7) << 7, _H - _CW)
        return pl.multiple_of(ra, 8), pl.multiple_of(ca, 128)

    def issue(j, slot):
        bb = g * per + j
        rr_, cr_, rf_, cf_ = coord[j]
        for cs, (r0, c0) in enumerate(((rr_, cr_), (rf_, cf_))):
            ra, ca = bases(r0, c0)
            for im, img in enumerate((fake_hbm, reala_hbm)):
                for c in range(3):
                    pltpu.make_async_copy(
                        img.at[bb, c, pl.ds(ra, _CHUNK), pl.ds(ca, _CW)],
                        buf_ref.at[slot, cs,
                                   pl.ds(im * 3 * _CHUNK + c * _CHUNK, _CHUNK)],
                        sem.at[slot]).start()

    for k in range(min(_DEPTH, per)):
        issue(k, k % _SLOTS)
    for j in range(per):
        slot = j % _SLOTS
        if j + _DEPTH < per:
            issue(j + _DEPTH, (j + _DEPTH) % _SLOTS)
        pltpu.make_async_copy(buf_ref.at[slot], buf_ref.at[slot],
                              sem.at[slot]).wait()
        rr_, cr_, rf_, cf_ = coord[j]
        crops = []
        for cs, (r0, c0) in enumerate(((rr_, cr_), (rf_, cf_))):
            ra, ca = bases(r0, c0)
            jc = lax.broadcasted_iota(jnp.int32, (_CW, _R), 0)
            kc = lax.broadcasted_iota(jnp.int32, (_CW, _R), 1)
            csel = (ca + jc == c0 + kc).astype(jnp.float32)
            ir = lax.broadcasted_iota(jnp.int32, (_R, _CHUNK), 0)
            jr = lax.broadcasted_iota(jnp.int32, (_R, _CHUNK), 1)
            rsel = ((ra + jr == r0 + ir)
                    & (r0 + ir <= _H - 1)).astype(jnp.float32)
            t = jnp.dot(buf_ref[slot, cs], csel,
                        preferred_element_type=jnp.float32)
            crops.append([
                jnp.dot(rsel, t[k * _CHUNK:(k + 1) * _CHUNK],
                        preferred_element_type=jnp.float32)
                for k in range(6)])
        fbr, rar = crops[0][:3], crops[0][3:]
        fbf, raf = crops[1][:3], crops[1][3:]
        for c in range(3):
            fbr_ref[j, c] = fbr[c]
            rar_ref[j, c] = rar[c]
            fbf_ref[j, c] = fbf[c]
            raf_ref[j, c] = raf[c]
            fabf_ref[j, c] = raf[c]
            fabf_ref[j, 3 + c] = fbf[c]
            rabr_ref[j, c] = rar[c]
            rabr_ref[j, 3 + c] = fbr[c]


def kernel(real_B, fake_B, real_A, score_map):
    del real_B
    B = score_map.shape[0]
    per = B // _GRID
    f32 = jnp.float32
    crop3 = jax.ShapeDtypeStruct((B, 3, _R, _R), f32)
    crop6 = jax.ShapeDtypeStruct((B, 6, _R, _R), f32)
    spec3 = pl.BlockSpec((per, 3, _R, _R), lambda g: (g, 0, 0, 0))
    spec6 = pl.BlockSpec((per, 6, _R, _R), lambda g: (g, 0, 0, 0))
    outs = pl.pallas_call(
        _propose_kernel,
        out_shape=(crop3, crop3, crop3, crop3, crop6, crop6),
        grid=(_GRID,),
        in_specs=[
            pl.BlockSpec((per, 1, _S, _S), lambda g: (g, 0, 0, 0)),
            pl.BlockSpec(memory_space=pl.ANY),
            pl.BlockSpec(memory_space=pl.ANY),
        ],
        out_specs=(spec3, spec3, spec3, spec3, spec6, spec6),
        scratch_shapes=[
            pltpu.VMEM((_SLOTS, 2, 2 * 3 * _CHUNK, _CW), f32),
            pltpu.SemaphoreType.DMA((_SLOTS,)),
        ],
        compiler_params=pltpu.CompilerParams(
            dimension_semantics=("parallel",),
        ),
        name="drpan_proposal",
    )(score_map, fake_B, real_A)
    return tuple(outs)


# speedup vs baseline: 2.6346x; 1.0136x over previous
import jax
import jax.numpy as jnp
from jax import lax
from jax.experimental import pallas as pl
from jax.experimental.pallas import tpu as pltpu

_R = 70
_H = 512
_S = 128
_STRIDE = 3
_CHUNK = 80
_CW = 256
_GRID = 2
_SLOTS = 6
_DEPTH = 5


def _propose_kernel(score_ref, fake_hbm, reala_hbm,
                    fbr_ref, rar_ref, fbf_ref, raf_ref, fabf_ref, rabr_ref,
                    buf_ref, sem):
    g = pl.program_id(0)
    per = fbr_ref.shape[0]

    s = score_ref[:, 0]
    ri = lax.broadcasted_iota(jnp.int32, (_S, _S), 0)
    ci = lax.broadcasted_iota(jnp.int32, (_S, _S), 1)
    flat = (ri * _S + ci)[None]
    big = jnp.int32(1 << 30)
    vmax = jnp.max(s, axis=(1, 2))
    vmin = jnp.min(s, axis=(1, 2))
    imax = jnp.min(jnp.where(s == vmax[:, None, None], flat, big), axis=(1, 2))
    imin = jnp.min(jnp.where(s == vmin[:, None, None], flat, big), axis=(1, 2))
    rr = jnp.where(vmax > 0.0, imax // _S, 0) * _STRIDE + _R
    cr = jnp.where(vmax > 0.0, imax % _S, 0) * _STRIDE + _R
    rf = jnp.where(vmin < 1.0, imin // _S, 1) * _STRIDE + _R
    cf = jnp.where(vmin < 1.0, imin % _S, 1) * _STRIDE + _R
    cm = jnp.stack([rr, cr, rf, cf], axis=0)
    coord = [[cm[q, j] for q in range(4)] for j in range(per)]

    def bases(r0, c0):
        ra = jnp.minimum((r0 >> 3) << 3, _H - _CHUNK)
        ca = jnp.minimum((c0 >> 7) << 7, _H - _CW)
        return pl.multiple_of(ra, 8), pl.multiple_of(ca, 128)

    def issue(j, slot):
        bb = g * per + j
        rr_, cr_, rf_, cf_ = coord[j]
        for cs, (r0, c0) in enumerate(((rr_, cr_), (rf_, cf_))):
            ra, ca = bases(r0, c0)
            for im, img in enumerate((fake_hbm, reala_hbm)):
                for c in range(3):
                    pltpu.make_async_copy(
                        img.at[bb, c, pl.ds(ra, _CHUNK), pl.ds(ca, _CW)],
                        buf_ref.at[slot, cs,
                                   pl.ds(im * 3 * _CHUNK + c * _CHUNK, _CHUNK)],
                        sem.at[slot]).start()

    for k in range(min(_DEPTH, per)):
        issue(k, k % _SLOTS)
    for j in range(per):
        slot = j % _SLOTS
        if j + _DEPTH < per:
            issue(j + _DEPTH, (j + _DEPTH) % _SLOTS)
        pltpu.make_async_copy(buf_ref.at[slot], buf_ref.at[slot],
                              sem.at[slot]).wait()
        rr_, cr_, rf_, cf_ = coord[j]
        crops = []
        for cs, (r0, c0) in enumerate(((rr_, cr_), (rf_, cf_))):
            ra, ca = bases(r0, c0)
            jc = lax.broadcasted_iota(jnp.int32, (_CW, _R), 0)
            kc = lax.broadcasted_iota(jnp.int32, (_CW, _R), 1)
            csel = (ca + jc == c0 + kc).astype(jnp.float32)
            t = jnp.dot(buf_ref[slot, cs], csel,
                        preferred_element_type=jnp.float32)
            rolled = pltpu.roll(t, -(r0 - ra), axis=0)
            i480 = lax.broadcasted_iota(jnp.int32, (6 * _CHUNK, _R), 0)
            valid = (i480 % _CHUNK) <= (_H - 1 - r0)
            tm = jnp.where(valid, rolled, 0.0)
            crops.append([tm[k * _CHUNK:k * _CHUNK + _R] for k in range(6)])
        fbr, rar = crops[0][:3], crops[0][3:]
        fbf, raf = crops[1][:3], crops[1][3:]
        for c in range(3):
            fbr_ref[j, c] = fbr[c]
            rar_ref[j, c] = rar[c]
            fbf_ref[j, c] = fbf[c]
            raf_ref[j, c] = raf[c]
            fabf_ref[j, c] = raf[c]
            fabf_ref[j, 3 + c] = fbf[c]
            rabr_ref[j, c] = rar[c]
            rabr_ref[j, 3 + c] = fbr[c]


def kernel(real_B, fake_B, real_A, score_map):
    del real_B
    B = score_map.shape[0]
    per = B // _GRID
    f32 = jnp.float32
    crop3 = jax.ShapeDtypeStruct((B, 3, _R, _R), f32)
    crop6 = jax.ShapeDtypeStruct((B, 6, _R, _R), f32)
    spec3 = pl.BlockSpec((per, 3, _R, _R), lambda g: (g, 0, 0, 0))
    spec6 = pl.BlockSpec((per, 6, _R, _R), lambda g: (g, 0, 0, 0))
    outs = pl.pallas_call(
        _propose_kernel,
        out_shape=(crop3, crop3, crop3, crop3, crop6, crop6),
        grid=(_GRID,),
        in_specs=[
            pl.BlockSpec((per, 1, _S, _S), lambda g: (g, 0, 0, 0)),
            pl.BlockSpec(memory_space=pl.ANY),
            pl.BlockSpec(memory_space=pl.ANY),
        ],
        out_specs=(spec3, spec3, spec3, spec3, spec6, spec6),
        scratch_shapes=[
            pltpu.VMEM((_SLOTS, 2, 2 * 3 * _CHUNK, _CW), f32),
            pltpu.SemaphoreType.DMA((_SLOTS,)),
        ],
        compiler_params=pltpu.CompilerParams(
            dimension_semantics=("parallel",),
        ),
        name="drpan_proposal",
    )(score_map, fake_B, real_A)
    return tuple(outs)
